# baseline (speedup 1.0000x reference)
import jax
import jax.numpy as jnp
from jax import lax
from jax.experimental import pallas as pl
from jax.experimental.pallas import tpu as pltpu
from jax.experimental.pallas import tpu_sc as plsc

_VOCAB = 30522
_NTOK = 100000
_RES = 8
_FEAT = 32
_FROZEN = 0
_BATCH = 16384
_FIELDS = 26

_NC = 2
_NS = 16
_NW = _NC * _NS

_S1_T = 128
_S1_NCH = (_NTOK + _S1_T - 1) // _S1_T
_S1_LAST = _S1_NCH - 1
_S1_TAIL = _NTOK - _S1_LAST * _S1_T
_S1_PH = 25
_NTOK_PAD = _S1_NCH * _S1_T
_S2_BT = _BATCH // 128


def _wid():
    return lax.axis_index("s") * _NC + lax.axis_index("c")


def _pool_body(rest_hbm, emb, pooled_k, i0, i1, r0, r1, t0, t1, o0, o1,
               si0, si1, sr0, sr1, so0, so1):
    w = _wid()
    IB, RB, TB, OB = (i0, i1), (r0, r1), (t0, t1), (o0, o1)
    SI, SR, SO = (si0, si1), (sr0, sr1), (so0, so1)
    iota = lax.iota(jnp.int32, 16)

    def chunk_of(it):
        return lax.min(w + it * _NW, _S1_LAST)

    def fire_ids(it, p):
        c = chunk_of(it)

        @pl.when(c < _S1_LAST)
        def _():
            pltpu.async_copy(
                rest_hbm.at[pl.ds(0, _RES), pl.ds(c * _S1_T, _S1_T)],
                IB[p], SI[p],
            )

        @pl.when(c == _S1_LAST)
        def _():
            pltpu.async_copy(
                rest_hbm.at[pl.ds(0, _RES), pl.ds(_S1_LAST * _S1_T, _S1_TAIL)],
                IB[p].at[pl.ds(0, _RES), pl.ds(0, _S1_TAIL)], SI[p],
            )

    def wait_ids(it, p):
        c = chunk_of(it)

        @pl.when(c < _S1_LAST)
        def _():
            pltpu.make_async_copy(
                rest_hbm.at[pl.ds(0, _RES), pl.ds(0, _S1_T)], IB[p], SI[p]
            ).wait()

        @pl.when(c == _S1_LAST)
        def _():
            pltpu.make_async_copy(
                rest_hbm.at[pl.ds(0, _RES), pl.ds(0, _S1_TAIL)],
                IB[p].at[pl.ds(0, _RES), pl.ds(0, _S1_TAIL)], SI[p],
            ).wait()

    def fire_streams(p):
        for j in range(_RES):
            pltpu.async_copy(
                emb.at[IB[p].at[j]], RB[p].at[pl.ds(128 * j, 128)], SR[p]
            )

    fire_ids(0, 0)
    fire_ids(1, 1)
    wait_ids(0, 0)
    fire_streams(0)

    def phase(q, p):
        it = 2 * q + p
        c = chunk_of(it)
        pltpu.make_async_copy(
            emb.at[IB[p].at[0]], RB[p], SR[p]
        ).wait()

        @pl.when(it < _S1_PH - 1)
        def _():
            wait_ids(it + 1, (p + 1) % 2)
            fire_streams((p + 1) % 2)

        @pl.when(it < _S1_PH - 2)
        def _():
            fire_ids(it + 2, p)

        @pl.when(it >= 2)
        def _():
            pltpu.make_async_copy(
                OB[p], pooled_k.at[pl.ds(0, _FEAT), pl.ds(0, _S1_T)], SO[p]
            ).wait()

        rows, tmp, out_v = RB[p], TB[p], OB[p]

        @plsc.parallel_loop(0, _S1_T, 2)
        def tok_body(t):
            for dt in range(2):
                tt = t + dt
                for h in range(2):
                    v = [rows[128 * r + tt, pl.ds(16 * h, 16)] for r in range(_RES)]
                    acc = ((v[0] + v[1]) + (v[2] + v[3])) + (
                        (v[4] + v[5]) + (v[6] + v[7])
                    )
                    tmp[tt, pl.ds(16 * h, 16)] = acc

        @plsc.parallel_loop(0, _S1_T // 16, 1)
        def tr_body(u):
            row_idx = u * 16 + iota
            for k in range(_FEAT):
                col = jnp.full((16,), k, jnp.int32)
                out_v[k, pl.ds(u * 16, 16)] = plsc.load_gather(
                    tmp, [row_idx, col]
                )
        pltpu.async_copy(
            out_v, pooled_k.at[pl.ds(0, _FEAT), pl.ds(c * _S1_T, _S1_T)], SO[p]
        )

    def pair(q, carry):
        phase(q, 0)
        phase(q, 1)
        return carry

    lax.fori_loop(0, _S1_PH // 2, pair, 0)
    phase(_S1_PH // 2, 0)
    for p in range(2):
        pltpu.make_async_copy(
            OB[p], pooled_k.at[pl.ds(0, _FEAT), pl.ds(0, _S1_T)], SO[p]
        ).wait()


def _lookup_body(baset_hbm, pooled_k, out5, pk_v, ib0, ib1, ib2, ib3,
                 ob0, ob1, si0, si1, si2, si3, so0, so1):
    w = _wid()
    kt = w // 8
    ks = w % 8
    IB, SI = (ib0, ib1, ib2, ib3), (si0, si1, si2, si3)
    OB, SO = (ob0, ob1), (so0, so1)
    pltpu.sync_copy(pooled_k.at[w], pk_v)

    def fire_ids(bt, p):
        pltpu.async_copy(
            baset_hbm.at[pl.ds(0, _FIELDS), pl.ds(bt * 128, 128)], IB[p], SI[p]
        )

    for p in range(4):
        fire_ids(p, p)

    def quad(q, carry):
        for p in range(4):
            bt = q * 4 + p
            po = p & 1
            pltpu.make_async_copy(
                baset_hbm.at[pl.ds(0, _FIELDS), pl.ds(0, 128)], IB[p], SI[p]
            ).wait()

            @pl.when(bt >= 2)
            def _():
                pltpu.make_async_copy(
                    OB[po],
                    out5.at[pl.ds(0, _FIELDS), pl.ds(0, 1), pl.ds(0, 1),
                            pl.ds(0, 1), :],
                    SO[po],
                ).wait()

            ids_v, out_v = IB[p], OB[po]

            @plsc.parallel_loop(0, _FIELDS, 13)
            def f_body(f2):
                for df in range(13):
                    f = f2 + df
                    for u in range(8):
                        ids = ids_v[f, pl.ds(u * 16, 16)]
                        out_v[f, 0, 0, 0, pl.ds(u * 16, 16)] = plsc.load_gather(
                            pk_v, [ids]
                        )
            pltpu.async_copy(
                out_v,
                out5.at[pl.ds(0, _FIELDS), pl.ds(kt, 1), pl.ds(bt, 1),
                        pl.ds(ks, 1), :],
                SO[po],
            )

            @pl.when(q < _S2_BT // 4 - 1)
            def _():
                fire_ids(bt + 4, p)

        return carry

    lax.fori_loop(0, _S2_BT // 4, quad, 0)
    for po in range(2):
        pltpu.make_async_copy(
            OB[po],
            out5.at[pl.ds(0, _FIELDS), pl.ds(0, 1), pl.ds(0, 1), pl.ds(0, 1), :],
            SO[po],
        ).wait()


def kernel(base_indices, reservoir_encoded, embedding):
    emb0 = embedding.at[_FROZEN].set(0.0)
    rest = reservoir_encoded.T
    baset = base_indices.T

    mesh = plsc.VectorSubcoreMesh(core_axis_name="c", subcore_axis_name="s")
    params = pltpu.CompilerParams(
        use_tc_tiling_on_sc=False, needs_layout_passes=False
    )

    pooled_k = pl.kernel(
        _pool_body,
        out_type=jax.ShapeDtypeStruct((_FEAT, _NTOK_PAD), jnp.float32),
        mesh=mesh,
        compiler_params=params,
        scratch_types=[
            pltpu.VMEM((_RES, _S1_T), jnp.int32),
            pltpu.VMEM((_RES, _S1_T), jnp.int32),
            pltpu.VMEM((_S1_T * _RES, _FEAT), jnp.float32),
            pltpu.VMEM((_S1_T * _RES, _FEAT), jnp.float32),
            pltpu.VMEM((_S1_T, _FEAT), jnp.float32),
            pltpu.VMEM((_S1_T, _FEAT), jnp.float32),
            pltpu.VMEM((_FEAT, _S1_T), jnp.float32),
            pltpu.VMEM((_FEAT, _S1_T), jnp.float32),
            pltpu.SemaphoreType.DMA,
            pltpu.SemaphoreType.DMA,
            pltpu.SemaphoreType.DMA,
            pltpu.SemaphoreType.DMA,
            pltpu.SemaphoreType.DMA,
            pltpu.SemaphoreType.DMA,
        ],
    )(rest, emb0)

    out5 = pl.kernel(
        _lookup_body,
        out_type=jax.ShapeDtypeStruct(
            (_FIELDS, _FEAT // 8, _BATCH // 128, 8, 128), jnp.float32
        ),
        mesh=mesh,
        compiler_params=params,
        scratch_types=[
            pltpu.VMEM((_NTOK_PAD,), jnp.float32),
            pltpu.VMEM((_FIELDS, 128), jnp.int32),
            pltpu.VMEM((_FIELDS, 128), jnp.int32),
            pltpu.VMEM((_FIELDS, 128), jnp.int32),
            pltpu.VMEM((_FIELDS, 128), jnp.int32),
            pltpu.VMEM((_FIELDS, 1, 1, 1, 128), jnp.float32),
            pltpu.VMEM((_FIELDS, 1, 1, 1, 128), jnp.float32),
            pltpu.SemaphoreType.DMA,
            pltpu.SemaphoreType.DMA,
            pltpu.SemaphoreType.DMA,
            pltpu.SemaphoreType.DMA,
            pltpu.SemaphoreType.DMA,
            pltpu.SemaphoreType.DMA,
        ],
    )(baset, pooled_k)

    return out5.transpose(2, 4, 0, 1, 3).reshape(_BATCH, _FIELDS, _FEAT)


# speedup vs baseline: 1.1962x; 1.1962x over previous
import jax
import jax.numpy as jnp
from jax import lax
from jax.experimental import pallas as pl
from jax.experimental.pallas import tpu as pltpu
from jax.experimental.pallas import tpu_sc as plsc

_VOCAB = 30522
_NTOK = 100000
_RES = 8
_FEAT = 32
_FROZEN = 0
_BATCH = 16384
_FIELDS = 26

_NC = 2
_NS = 16
_NW = _NC * _NS

_S1_T = 128
_S1_NCH = (_NTOK + _S1_T - 1) // _S1_T
_S1_LAST = _S1_NCH - 1
_S1_TAIL = _NTOK - _S1_LAST * _S1_T
_S1_PH = 25
_NTOK_PAD = _S1_NCH * _S1_T
_S2_BT = _BATCH // 128


def _wid():
    return lax.axis_index("s") * _NC + lax.axis_index("c")


def _pool_body(rest_hbm, emb, pooled_k, i0, i1, r0, r1, t0, t1, o0, o1,
               si0, si1, sr0, sr1, so0, so1):
    w = _wid()
    IB, RB, TB, OB = (i0, i1), (r0, r1), (t0, t1), (o0, o1)
    SI, SR, SO = (si0, si1), (sr0, sr1), (so0, so1)
    iota = lax.iota(jnp.int32, 16)

    def chunk_of(it):
        return lax.min(w + it * _NW, _S1_LAST)

    def fire_ids(it, p):
        c = chunk_of(it)

        @pl.when(c < _S1_LAST)
        def _():
            pltpu.async_copy(
                rest_hbm.at[pl.ds(0, _RES), pl.ds(c * _S1_T, _S1_T)],
                IB[p], SI[p],
            )

        @pl.when(c == _S1_LAST)
        def _():
            pltpu.async_copy(
                rest_hbm.at[pl.ds(0, _RES), pl.ds(_S1_LAST * _S1_T, _S1_TAIL)],
                IB[p].at[pl.ds(0, _RES), pl.ds(0, _S1_TAIL)], SI[p],
            )

    def wait_ids(it, p):
        c = chunk_of(it)

        @pl.when(c < _S1_LAST)
        def _():
            pltpu.make_async_copy(
                rest_hbm.at[pl.ds(0, _RES), pl.ds(0, _S1_T)], IB[p], SI[p]
            ).wait()

        @pl.when(c == _S1_LAST)
        def _():
            pltpu.make_async_copy(
                rest_hbm.at[pl.ds(0, _RES), pl.ds(0, _S1_TAIL)],
                IB[p].at[pl.ds(0, _RES), pl.ds(0, _S1_TAIL)], SI[p],
            ).wait()

    def fire_streams(p):
        for j in range(_RES):
            pltpu.async_copy(
                emb.at[IB[p].at[j]], RB[p].at[pl.ds(128 * j, 128)], SR[p]
            )

    fire_ids(0, 0)
    fire_ids(1, 1)
    wait_ids(0, 0)
    fire_streams(0)

    def phase(q, p):
        it = 2 * q + p
        c = chunk_of(it)
        pltpu.make_async_copy(
            emb.at[IB[p].at[0]], RB[p], SR[p]
        ).wait()

        @pl.when(it < _S1_PH - 1)
        def _():
            wait_ids(it + 1, (p + 1) % 2)
            fire_streams((p + 1) % 2)

        @pl.when(it < _S1_PH - 2)
        def _():
            fire_ids(it + 2, p)

        @pl.when(it >= 2)
        def _():
            pltpu.make_async_copy(
                OB[p], pooled_k.at[pl.ds(0, _FEAT), pl.ds(0, _S1_T)], SO[p]
            ).wait()

        rows, tmp, out_v = RB[p], TB[p], OB[p]

        @plsc.parallel_loop(0, _S1_T, 2)
        def tok_body(t):
            for dt in range(2):
                tt = t + dt
                for h in range(2):
                    v = [rows[128 * r + tt, pl.ds(16 * h, 16)] for r in range(_RES)]
                    acc = ((v[0] + v[1]) + (v[2] + v[3])) + (
                        (v[4] + v[5]) + (v[6] + v[7])
                    )
                    tmp[tt, pl.ds(16 * h, 16)] = acc

        @plsc.parallel_loop(0, _S1_T // 16, 1)
        def tr_body(u):
            row_idx = u * 16 + iota
            for k in range(_FEAT):
                col = jnp.full((16,), k, jnp.int32)
                out_v[k, pl.ds(u * 16, 16)] = plsc.load_gather(
                    tmp, [row_idx, col]
                )
        pltpu.async_copy(
            out_v, pooled_k.at[pl.ds(0, _FEAT), pl.ds(c * _S1_T, _S1_T)], SO[p]
        )

    def pair(q, carry):
        phase(q, 0)
        phase(q, 1)
        return carry

    lax.fori_loop(0, _S1_PH // 2, pair, 0)
    phase(_S1_PH // 2, 0)
    for p in range(2):
        pltpu.make_async_copy(
            OB[p], pooled_k.at[pl.ds(0, _FEAT), pl.ds(0, _S1_T)], SO[p]
        ).wait()


def _lookup_body(baset_hbm, pooled_k, out5, pk_v, ib0, ib1, ib2, ib3,
                 ob0, ob1, si0, si1, si2, si3, so0, so1):
    w = _wid()
    kt = w // 8
    ks = w % 8
    IB, SI = (ib0, ib1, ib2, ib3), (si0, si1, si2, si3)
    OB, SO = (ob0, ob1), (so0, so1)
    pltpu.sync_copy(pooled_k.at[w], pk_v)

    def fire_ids(bt, p):
        pltpu.async_copy(
            baset_hbm.at[pl.ds(0, _FIELDS), pl.ds(bt * 128, 128)], IB[p], SI[p]
        )

    for p in range(4):
        fire_ids(p, p)

    def quad(q, carry):
        for p in range(4):
            bt = q * 4 + p
            po = p & 1
            pltpu.make_async_copy(
                baset_hbm.at[pl.ds(0, _FIELDS), pl.ds(0, 128)], IB[p], SI[p]
            ).wait()

            @pl.when(bt >= 2)
            def _():
                pltpu.make_async_copy(
                    OB[po],
                    out5.at[pl.ds(0, _FIELDS), pl.ds(0, 1), pl.ds(0, 1),
                            pl.ds(0, 1), :],
                    SO[po],
                ).wait()

            ids_v, out_v = IB[p], OB[po]

            @plsc.parallel_loop(0, _FIELDS, 2)
            def f_body(f2):
                for df in range(2):
                    f = f2 + df
                    for u in range(8):
                        ids = ids_v[f, pl.ds(u * 16, 16)]
                        out_v[f, 0, 0, 0, pl.ds(u * 16, 16)] = plsc.load_gather(
                            pk_v, [ids]
                        )
            pltpu.async_copy(
                out_v,
                out5.at[pl.ds(0, _FIELDS), pl.ds(kt, 1), pl.ds(bt, 1),
                        pl.ds(ks, 1), :],
                SO[po],
            )

            @pl.when(q < _S2_BT // 4 - 1)
            def _():
                fire_ids(bt + 4, p)

        return carry

    lax.fori_loop(0, _S2_BT // 4, quad, 0)
    for po in range(2):
        pltpu.make_async_copy(
            OB[po],
            out5.at[pl.ds(0, _FIELDS), pl.ds(0, 1), pl.ds(0, 1), pl.ds(0, 1), :],
            SO[po],
        ).wait()


def kernel(base_indices, reservoir_encoded, embedding):
    emb0 = embedding.at[_FROZEN].set(0.0)
    rest = reservoir_encoded.T
    baset = base_indices.T

    mesh = plsc.VectorSubcoreMesh(core_axis_name="c", subcore_axis_name="s")
    params = pltpu.CompilerParams(
        use_tc_tiling_on_sc=False, needs_layout_passes=False
    )

    pooled_k = pl.kernel(
        _pool_body,
        out_type=jax.ShapeDtypeStruct((_FEAT, _NTOK_PAD), jnp.float32),
        mesh=mesh,
        compiler_params=params,
        scratch_types=[
            pltpu.VMEM((_RES, _S1_T), jnp.int32),
            pltpu.VMEM((_RES, _S1_T), jnp.int32),
            pltpu.VMEM((_S1_T * _RES, _FEAT), jnp.float32),
            pltpu.VMEM((_S1_T * _RES, _FEAT), jnp.float32),
            pltpu.VMEM((_S1_T, _FEAT), jnp.float32),
            pltpu.VMEM((_S1_T, _FEAT), jnp.float32),
            pltpu.VMEM((_FEAT, _S1_T), jnp.float32),
            pltpu.VMEM((_FEAT, _S1_T), jnp.float32),
            pltpu.SemaphoreType.DMA,
            pltpu.SemaphoreType.DMA,
            pltpu.SemaphoreType.DMA,
            pltpu.SemaphoreType.DMA,
            pltpu.SemaphoreType.DMA,
            pltpu.SemaphoreType.DMA,
        ],
    )(rest, emb0)

    out5 = pl.kernel(
        _lookup_body,
        out_type=jax.ShapeDtypeStruct(
            (_FIELDS, _FEAT // 8, _BATCH // 128, 8, 128), jnp.float32
        ),
        mesh=mesh,
        compiler_params=params,
        scratch_types=[
            pltpu.VMEM((_NTOK_PAD,), jnp.float32),
            pltpu.VMEM((_FIELDS, 128), jnp.int32),
            pltpu.VMEM((_FIELDS, 128), jnp.int32),
            pltpu.VMEM((_FIELDS, 128), jnp.int32),
            pltpu.VMEM((_FIELDS, 128), jnp.int32),
            pltpu.VMEM((_FIELDS, 1, 1, 1, 128), jnp.float32),
            pltpu.VMEM((_FIELDS, 1, 1, 1, 128), jnp.float32),
            pltpu.SemaphoreType.DMA,
            pltpu.SemaphoreType.DMA,
            pltpu.SemaphoreType.DMA,
            pltpu.SemaphoreType.DMA,
            pltpu.SemaphoreType.DMA,
            pltpu.SemaphoreType.DMA,
        ],
    )(baset, pooled_k)

    return out5.transpose(2, 4, 0, 1, 3).reshape(_BATCH, _FIELDS, _FEAT)
